# Initial kernel scaffold; baseline (speedup 1.0000x reference)
import jax
import jax.numpy as jnp
from jax.experimental import pallas as pl


def kernel(points, w1, b1, g1, be1, w2, b2, g2, be2, w3, b3):
    raise NotImplementedError("write your pallas kernel here")


# speedup vs baseline: 1.5324x; 1.5324x over previous
import functools

import jax
import jax.numpy as jnp
from jax import lax
from jax.experimental import pallas as pl
from jax.experimental.pallas import tpu as pltpu
from jax.experimental.pallas import tpu_sc as plsc

BEV_H = 200
BEV_W = 200
RES = 0.5
GX0 = -50.0
GY0 = -50.0

NPTS = 20000
NG = 8
NT = 4
ROWS = BEV_H // NT
CELLS = ROWS * BEV_W
PPG = NPTS // NG
CHUNKS = (PPG + 15) // 16
PPGP = CHUNKS * 16
NCH = 6
HW = BEV_H * BEV_W
PAD = BEV_W + 1
BIGKEY = 1 << 28


def _take16(x, idx):
    return lax.gather(
        x, idx[:, None],
        dimension_numbers=lax.GatherDimensionNumbers(
            offset_dims=(), collapsed_slice_dims=(0,), start_index_map=(0,)),
        slice_sizes=(1,), mode=lax.GatherScatterMode.PROMISE_IN_BOUNDS)


def _sort16(key, val):
    return plsc.sort_key_val(key, val)


def _bin_resolve(xv, yv, zv, iv, lane_valid, sub):
    iota = lax.iota(jnp.int32, 16)
    xi = jnp.clip(((xv - GX0) * 2.0).astype(jnp.int32), 0, BEV_W - 1)
    yi = jnp.clip(((yv - GY0) * 2.0).astype(jnp.int32), 0, BEV_H - 1)
    h = ((zv > -2.0).astype(jnp.int32) + (zv > 0.0).astype(jnp.int32)
         + (zv > 2.0).astype(jnp.int32) + (zv > 4.0).astype(jnp.int32))
    h = jnp.minimum(h, 3)
    y0 = sub * ROWS
    mine = lane_valid & (yi >= y0) & (yi < y0 + ROWS)
    cell = (yi - y0) * BEV_W + xi
    key = jnp.where(mine, cell * 8 + h, jnp.int32(BIGKEY))

    skey, perm = _sort16(key, iota)
    scell = jnp.right_shift(skey, 3)
    val_s = _take16(zv + 2.0, perm)

    hval = val_s
    lane_m = perm
    for s in (1, 2, 4, 8):
        src = jnp.maximum(iota - s, 0)
        k2 = _take16(skey, src)
        hval = jnp.where(k2 == skey,
                         jnp.maximum(hval, _take16(hval, src)), hval)
        lane_m = jnp.where(jnp.right_shift(k2, 3) == scell,
                           jnp.maximum(lane_m, _take16(lane_m, src)), lane_m)

    nxt = jnp.minimum(iota + 1, 15)
    kn = _take16(skey, nxt)
    end_h = (iota == 15) | (kn != skey)
    end_c = (iota == 15) | (jnp.right_shift(kn, 3) != scell)
    mine_s = skey < BIGKEY
    mask_h = end_h & mine_s
    mask_c = end_c & mine_s

    ch_s = jnp.bitwise_and(skey, 7)
    cell_s = jnp.where(mine_s, scell, 0)
    ival = jnp.maximum(_take16(iv, lane_m), 0.0)
    return ch_s, cell_s, hval, mask_h, ival, mask_c


def _sc_body(pts_hbm, out_hbm, pts_v, slab):
    wid = lax.axis_index("c") * 16 + lax.axis_index("s")
    g = wid // NT
    sub = wid % NT

    pltpu.sync_copy(pts_hbm.at[g], pts_v)

    zeros = jnp.zeros((16,), jnp.float32)

    def zero_body(i, _):
        for c in range(NCH):
            slab[pl.ds(c * CELLS + i * 16, 16)] = zeros
        return 0

    lax.fori_loop(0, CELLS // 16, zero_body, 0)

    ones = jnp.ones((16,), jnp.float32)
    iota = lax.iota(jnp.int32, 16)

    def body(ci, _):
        off = ci * 16
        xv = pts_v[0, pl.ds(off, 16)]
        yv = pts_v[1, pl.ds(off, 16)]
        zv = pts_v[2, pl.ds(off, 16)]
        iv = pts_v[3, pl.ds(off, 16)]
        lane_valid = (off + iota) < PPG
        ch_s, cell_s, hval, mask_h, ival, mask_c = _bin_resolve(
            xv, yv, zv, iv, lane_valid, sub)
        addr = ch_s * CELLS + cell_s
        cur = plsc.load_gather(slab, [addr], mask=mask_h)
        plsc.store_scatter(slab, [addr],
                           jnp.maximum(cur, hval), mask=mask_h)
        plsc.store_scatter(slab, [4 * CELLS + cell_s], ival, mask=mask_c)
        plsc.store_scatter(slab, [5 * CELLS + cell_s], ones, mask=mask_c)
        return 0

    lax.fori_loop(0, CHUNKS, body, 0)

    for c in range(NCH):
        pltpu.sync_copy(slab.at[pl.ds(c * CELLS, CELLS)],
                        out_hbm.at[sub, c, g])


def _sc_build(pts):
    mesh = plsc.VectorSubcoreMesh(
        core_axis_name="c", subcore_axis_name="s", num_cores=2,
        num_subcores=16)
    f = pl.kernel(
        _sc_body,
        out_type=jax.ShapeDtypeStruct((NT, NCH, NG, CELLS), jnp.float32),
        mesh=mesh,
        scratch_types=[
            pltpu.VMEM((4, PPGP), jnp.float32),
            pltpu.VMEM((NCH * CELLS,), jnp.float32),
        ],
        compiler_params=pltpu.CompilerParams(
            use_tc_tiling_on_sc=False, needs_layout_passes=False),
    )
    return f(pts)


PADC = 256
BLK = 8192
NBF = 5
TOTW = NBF * BLK
COLS = PADC + TOTW + PADC


def _dot(a, b):
    return jax.lax.dot_general(
        a, b, (((1,), (0,)), ((), ())),
        preferred_element_type=jnp.float32,
        precision=jax.lax.Precision.HIGHEST)


def _blk_masks(b):
    col = b * BLK + lax.broadcasted_iota(jnp.int32, (1, BLK), 1)
    inrow = lax.rem(col, BEV_W)
    m_l = (inrow != 0).astype(jnp.float32)
    m_r = (inrow != BEV_W - 1).astype(jnp.float32)
    valid = (col < HW).astype(jnp.float32)
    return m_l, m_r, valid


def _conv_chunk(pad_ref, wr, b, m_l, m_r):
    start = pl.multiple_of(b * BLK, 128)
    chunk = pad_ref[:, pl.ds(start, BLK + 2 * PADC)]
    acc = None
    for dy in range(3):
        for dx in range(3):
            off = (dy - 1) * BEV_W + (dx - 1)
            sft = chunk[:, PADC + off:PADC + off + BLK]
            if dx == 0:
                sft = sft * m_r
            elif dx == 2:
                sft = sft * m_l
            d = _dot(wr[dy, dx], sft)
            acc = d if acc is None else acc + d
    return acc


def _tc1_body(part, w1r, g1r, be1r, w2r, f2_ref, m2_ref, v2_ref, pad1, pad2):
    pad1[:, :PADC] = jnp.zeros((5, PADC), jnp.float32)
    pad1[:, PADC + HW:] = jnp.zeros((5, COLS - PADC - HW), jnp.float32)
    pad2[:, :PADC] = jnp.zeros((32, PADC), jnp.float32)

    for sub in range(NT):
        dst = pl.ds(PADC + sub * CELLS, CELLS)
        for c in range(4):
            pad1[pl.ds(c, 1), dst] = jnp.max(part[sub, c], axis=0,
                                             keepdims=True)
        v8 = part[sub, 4]
        f8 = part[sub, 5]
        g8 = lax.broadcasted_iota(jnp.int32, (NG, CELLS), 0)
        key = jnp.where(f8 > 0.0, g8, -1)
        kmax = jnp.max(key, axis=0, keepdims=True)
        r = jnp.sum(jnp.where((key == kmax) & (key >= 0), v8,
                              jnp.zeros_like(v8)), axis=0, keepdims=True)
        pad1[pl.ds(4, 1), dst] = r

    def conv1_blk(b, _):
        m_l, m_r, _ = _blk_masks(b)
        dst = pl.multiple_of(PADC + b * BLK, 128)
        pad2[:, pl.ds(dst, BLK)] = _conv_chunk(pad1, w1r, b, m_l, m_r)
        return 0

    lax.fori_loop(0, NBF, conv1_blk, 0)

    def sum1_blk(b, tot):
        _, _, valid = _blk_masks(b)
        src = pl.multiple_of(PADC + b * BLK, 128)
        return tot + jnp.sum(pad2[:, pl.ds(src, BLK)] * valid, axis=1,
                             keepdims=True)

    mean = lax.fori_loop(0, NBF, sum1_blk,
                         jnp.zeros((32, 1), jnp.float32)) / HW

    def var1_blk(b, tot):
        _, _, valid = _blk_masks(b)
        src = pl.multiple_of(PADC + b * BLK, 128)
        d = (pad2[:, pl.ds(src, BLK)] - mean) * valid
        return tot + jnp.sum(d * d, axis=1, keepdims=True)

    var = lax.fori_loop(0, NBF, var1_blk,
                        jnp.zeros((32, 1), jnp.float32)) / HW

    def norm1_blk(b, _):
        src = pl.multiple_of(PADC + b * BLK, 128)
        blk = pad2[:, pl.ds(src, BLK)]
        y = g1r[...] * (blk - mean) / jnp.sqrt(var + 1e-5) + be1r[...]
        pad2[:, pl.ds(src, BLK)] = jnp.maximum(y, 0.0)
        return 0

    lax.fori_loop(0, NBF, norm1_blk, 0)
    pad2[:, PADC + HW:] = jnp.zeros((32, COLS - PADC - HW), jnp.float32)

    def conv2_blk(b, _):
        m_l, m_r, _ = _blk_masks(b)
        dst = pl.multiple_of(b * BLK, 128)
        f2_ref[:, pl.ds(dst, BLK)] = _conv_chunk(pad2, w2r, b, m_l, m_r)
        return 0

    lax.fori_loop(0, NBF, conv2_blk, 0)

    def sum2_blk(b, tot):
        _, _, valid = _blk_masks(b)
        src = pl.multiple_of(b * BLK, 128)
        return tot + jnp.sum(f2_ref[:, pl.ds(src, BLK)] * valid, axis=1,
                             keepdims=True)

    m2 = lax.fori_loop(0, NBF, sum2_blk,
                       jnp.zeros((64, 1), jnp.float32)) / HW
    m2_ref[...] = m2

    def var2_blk(b, tot):
        _, _, valid = _blk_masks(b)
        src = pl.multiple_of(b * BLK, 128)
        d = (f2_ref[:, pl.ds(src, BLK)] - m2) * valid
        return tot + jnp.sum(d * d, axis=1, keepdims=True)

    v2_ref[...] = lax.fori_loop(0, NBF, var2_blk,
                                jnp.zeros((64, 1), jnp.float32)) / HW


def _tc2_body(f2_in, g2r, be2r, m2r, v2r, w3r, b3r, out_ref):
    def blk(b, _):
        src = pl.multiple_of(b * BLK, 128)
        x = f2_in[:, pl.ds(src, BLK)]
        y = g2r[...] * (x - m2r[...]) / jnp.sqrt(v2r[...] + 1e-5) + be2r[...]
        out_ref[:, pl.ds(src, BLK)] = (
            _dot(w3r[...], jnp.maximum(y, 0.0)) + b3r[...])
        return 0

    lax.fori_loop(0, NBF, blk, 0)


def _tc_encode(part, w1, b1, g1, be1, w2, b2, g2, be2, w3, b3):
    f2, m2, v2 = pl.pallas_call(
        _tc1_body,
        out_shape=[
            jax.ShapeDtypeStruct((64, TOTW), jnp.float32),
            jax.ShapeDtypeStruct((64, 1), jnp.float32),
            jax.ShapeDtypeStruct((64, 1), jnp.float32),
        ],
        scratch_shapes=[
            pltpu.VMEM((5, COLS), jnp.float32),
            pltpu.VMEM((32, COLS), jnp.float32),
        ],
    )(part, w1, g1, be1, w2)
    return pl.pallas_call(
        _tc2_body,
        out_shape=jax.ShapeDtypeStruct((128, TOTW), jnp.float32),
    )(f2, g2, be2, m2, v2, w3, b3)


def kernel(points, w1, b1, g1, be1, w2, b2, g2, be2, w3, b3):
    pts = points.T.reshape(4, NG, PPG)
    pts = jnp.pad(pts, ((0, 0), (0, 0), (0, PPGP - PPG)))
    pts = pts.transpose(1, 0, 2)

    part = _sc_build(pts)

    out = _tc_encode(
        part,
        w1.transpose(2, 3, 0, 1),
        b1.reshape(32, 1), g1.reshape(32, 1), be1.reshape(32, 1),
        w2.transpose(2, 3, 0, 1),
        b2.reshape(64, 1), g2.reshape(64, 1), be2.reshape(64, 1),
        w3.reshape(128, 64), b3.reshape(128, 1))
    return out[:, :HW].reshape(1, 128, BEV_H, BEV_W)


# speedup vs baseline: 2.7051x; 1.7653x over previous
import functools

import jax
import jax.numpy as jnp
from jax import lax
from jax.experimental import pallas as pl
from jax.experimental.pallas import tpu as pltpu
from jax.experimental.pallas import tpu_sc as plsc

BEV_H = 200
BEV_W = 200
RES = 0.5
GX0 = -50.0
GY0 = -50.0

NPTS = 20000
NG = 8
NT = 4
ROWS = BEV_H // NT
CELLS = ROWS * BEV_W
PPG = NPTS // NG
CHUNKS = (PPG + 15) // 16
PPGP = CHUNKS * 16
NCH = 6
HW = BEV_H * BEV_W
PAD = BEV_W + 1
BIGKEY = 1 << 28


def _take16(x, idx):
    return lax.gather(
        x, idx[:, None],
        dimension_numbers=lax.GatherDimensionNumbers(
            offset_dims=(), collapsed_slice_dims=(0,), start_index_map=(0,)),
        slice_sizes=(1,), mode=lax.GatherScatterMode.PROMISE_IN_BOUNDS)


def _sort16(key, val):
    return plsc.sort_key_val(key, val)


def _bin_resolve(xv, yv, zv, iv, lane_valid, sub):
    iota = lax.iota(jnp.int32, 16)
    xi = jnp.clip(((xv - GX0) * 2.0).astype(jnp.int32), 0, BEV_W - 1)
    yi = jnp.clip(((yv - GY0) * 2.0).astype(jnp.int32), 0, BEV_H - 1)
    h = ((zv > -2.0).astype(jnp.int32) + (zv > 0.0).astype(jnp.int32)
         + (zv > 2.0).astype(jnp.int32) + (zv > 4.0).astype(jnp.int32))
    h = jnp.minimum(h, 3)
    y0 = sub * ROWS
    mine = lane_valid & (yi >= y0) & (yi < y0 + ROWS)
    cell = (yi - y0) * BEV_W + xi
    key = jnp.where(mine, cell * 8 + h, jnp.int32(BIGKEY))

    skey, perm = _sort16(key, iota)
    scell = jnp.right_shift(skey, 3)
    val_s = _take16(zv + 2.0, perm)

    hval = val_s
    lane_m = perm
    for s in (1, 2, 4, 8):
        src = jnp.maximum(iota - s, 0)
        k2 = _take16(skey, src)
        hval = jnp.where(k2 == skey,
                         jnp.maximum(hval, _take16(hval, src)), hval)
        lane_m = jnp.where(jnp.right_shift(k2, 3) == scell,
                           jnp.maximum(lane_m, _take16(lane_m, src)), lane_m)

    nxt = jnp.minimum(iota + 1, 15)
    kn = _take16(skey, nxt)
    end_h = (iota == 15) | (kn != skey)
    end_c = (iota == 15) | (jnp.right_shift(kn, 3) != scell)
    mine_s = skey < BIGKEY
    mask_h = end_h & mine_s
    mask_c = end_c & mine_s

    ch_s = jnp.bitwise_and(skey, 7)
    cell_s = jnp.where(mine_s, scell, 0)
    ival = jnp.maximum(_take16(iv, lane_m), 0.0)
    return ch_s, cell_s, hval, mask_h, ival, mask_c


def _sc_body(pts_hbm, out_hbm, pts_v, slab):
    wid = lax.axis_index("c") * 16 + lax.axis_index("s")
    g = wid // NT
    sub = wid % NT

    pltpu.sync_copy(pts_hbm.at[g], pts_v)

    zeros = jnp.zeros((16,), jnp.float32)

    def zero_body(i, _):
        for c in range(NCH):
            slab[pl.ds(c * CELLS + i * 16, 16)] = zeros
        return 0

    lax.fori_loop(0, CELLS // 16, zero_body, 0)

    ones = jnp.ones((16,), jnp.float32)
    iota = lax.iota(jnp.int32, 16)

    def body(ci, _):
        off = ci * 16
        xv = pts_v[0, pl.ds(off, 16)]
        yv = pts_v[1, pl.ds(off, 16)]
        zv = pts_v[2, pl.ds(off, 16)]
        iv = pts_v[3, pl.ds(off, 16)]
        lane_valid = (off + iota) < PPG
        ch_s, cell_s, hval, mask_h, ival, mask_c = _bin_resolve(
            xv, yv, zv, iv, lane_valid, sub)
        addr = ch_s * CELLS + cell_s
        cur = plsc.load_gather(slab, [addr], mask=mask_h)
        plsc.store_scatter(slab, [addr],
                           jnp.maximum(cur, hval), mask=mask_h)
        plsc.store_scatter(slab, [4 * CELLS + cell_s], ival, mask=mask_c)
        plsc.store_scatter(slab, [5 * CELLS + cell_s], ones, mask=mask_c)
        return 0

    lax.fori_loop(0, CHUNKS, body, 0)

    for c in range(NCH):
        pltpu.sync_copy(slab.at[pl.ds(c * CELLS, CELLS)],
                        out_hbm.at[sub, c, g])


def _sc_build(pts):
    mesh = plsc.VectorSubcoreMesh(
        core_axis_name="c", subcore_axis_name="s", num_cores=2,
        num_subcores=16)
    f = pl.kernel(
        _sc_body,
        out_type=jax.ShapeDtypeStruct((NT, NCH, NG, CELLS), jnp.float32),
        mesh=mesh,
        scratch_types=[
            pltpu.VMEM((4, PPGP), jnp.float32),
            pltpu.VMEM((NCH * CELLS,), jnp.float32),
        ],
        compiler_params=pltpu.CompilerParams(
            use_tc_tiling_on_sc=False, needs_layout_passes=False),
    )
    return f(pts)


PADC = 256
BLK = 8192
NBF = 5
TOTW = NBF * BLK
COLS = PADC + TOTW + PADC


def _dot(a, b):
    return jax.lax.dot_general(
        a, b, (((1,), (0,)), ((), ())),
        preferred_element_type=jnp.float32,
        precision=jax.lax.Precision.DEFAULT)


def _blk_masks(b):
    col = b * BLK + lax.broadcasted_iota(jnp.int32, (1, BLK), 1)
    inrow = lax.rem(col, BEV_W)
    m_l = (inrow != 0).astype(jnp.float32)
    m_r = (inrow != BEV_W - 1).astype(jnp.float32)
    valid = (col < HW).astype(jnp.float32)
    return m_l, m_r, valid


def _conv_chunk(pad_ref, wr, b, m_l, m_r):
    start = pl.multiple_of(b * BLK, 128)
    chunk = pad_ref[:, pl.ds(start, BLK + 2 * PADC)]
    acc = None
    for dy in range(3):
        for dx in range(3):
            off = (dy - 1) * BEV_W + (dx - 1)
            sft = chunk[:, PADC + off:PADC + off + BLK]
            if dx == 0:
                sft = sft * m_r
            elif dx == 2:
                sft = sft * m_l
            d = _dot(wr[dy, dx], sft)
            acc = d if acc is None else acc + d
    return acc


def _tc1_body(part, w1r, g1r, be1r, w2r, f2_ref, m2_ref, v2_ref, pad1, pad2):
    pad1[:, :PADC] = jnp.zeros((5, PADC), jnp.float32)
    pad1[:, PADC + HW:] = jnp.zeros((5, COLS - PADC - HW), jnp.float32)
    pad2[:, :PADC] = jnp.zeros((32, PADC), jnp.float32)

    for sub in range(NT):
        dst = pl.ds(PADC + sub * CELLS, CELLS)
        for c in range(4):
            pad1[pl.ds(c, 1), dst] = jnp.max(part[sub, c], axis=0,
                                             keepdims=True)
        v8 = part[sub, 4]
        f8 = part[sub, 5]
        g8 = lax.broadcasted_iota(jnp.int32, (NG, CELLS), 0)
        key = jnp.where(f8 > 0.0, g8, -1)
        kmax = jnp.max(key, axis=0, keepdims=True)
        r = jnp.sum(jnp.where((key == kmax) & (key >= 0), v8,
                              jnp.zeros_like(v8)), axis=0, keepdims=True)
        pad1[pl.ds(4, 1), dst] = r

    def conv1_blk(b, _):
        m_l, m_r, _ = _blk_masks(b)
        dst = pl.multiple_of(PADC + b * BLK, 128)
        pad2[:, pl.ds(dst, BLK)] = _conv_chunk(pad1, w1r, b, m_l, m_r)
        return 0

    lax.fori_loop(0, NBF, conv1_blk, 0)

    def sum1_blk(b, tot):
        _, _, valid = _blk_masks(b)
        src = pl.multiple_of(PADC + b * BLK, 128)
        return tot + jnp.sum(pad2[:, pl.ds(src, BLK)] * valid, axis=1,
                             keepdims=True)

    mean = lax.fori_loop(0, NBF, sum1_blk,
                         jnp.zeros((32, 1), jnp.float32)) / HW

    def var1_blk(b, tot):
        _, _, valid = _blk_masks(b)
        src = pl.multiple_of(PADC + b * BLK, 128)
        d = (pad2[:, pl.ds(src, BLK)] - mean) * valid
        return tot + jnp.sum(d * d, axis=1, keepdims=True)

    var = lax.fori_loop(0, NBF, var1_blk,
                        jnp.zeros((32, 1), jnp.float32)) / HW

    def norm1_blk(b, _):
        src = pl.multiple_of(PADC + b * BLK, 128)
        blk = pad2[:, pl.ds(src, BLK)]
        y = g1r[...] * (blk - mean) / jnp.sqrt(var + 1e-5) + be1r[...]
        pad2[:, pl.ds(src, BLK)] = jnp.maximum(y, 0.0)
        return 0

    lax.fori_loop(0, NBF, norm1_blk, 0)
    pad2[:, PADC + HW:] = jnp.zeros((32, COLS - PADC - HW), jnp.float32)

    def conv2_blk(b, _):
        m_l, m_r, _ = _blk_masks(b)
        dst = pl.multiple_of(b * BLK, 128)
        f2_ref[:, pl.ds(dst, BLK)] = _conv_chunk(pad2, w2r, b, m_l, m_r)
        return 0

    lax.fori_loop(0, NBF, conv2_blk, 0)

    def sum2_blk(b, tot):
        _, _, valid = _blk_masks(b)
        src = pl.multiple_of(b * BLK, 128)
        return tot + jnp.sum(f2_ref[:, pl.ds(src, BLK)] * valid, axis=1,
                             keepdims=True)

    m2 = lax.fori_loop(0, NBF, sum2_blk,
                       jnp.zeros((64, 1), jnp.float32)) / HW
    m2_ref[...] = m2

    def var2_blk(b, tot):
        _, _, valid = _blk_masks(b)
        src = pl.multiple_of(b * BLK, 128)
        d = (f2_ref[:, pl.ds(src, BLK)] - m2) * valid
        return tot + jnp.sum(d * d, axis=1, keepdims=True)

    v2_ref[...] = lax.fori_loop(0, NBF, var2_blk,
                                jnp.zeros((64, 1), jnp.float32)) / HW


def _tc2_body(f2_in, g2r, be2r, m2r, v2r, w3r, b3r, out_ref):
    def blk(b, _):
        src = pl.multiple_of(b * BLK, 128)
        x = f2_in[:, pl.ds(src, BLK)]
        y = g2r[...] * (x - m2r[...]) / jnp.sqrt(v2r[...] + 1e-5) + be2r[...]
        out_ref[:, pl.ds(src, BLK)] = (
            _dot(w3r[...], jnp.maximum(y, 0.0)) + b3r[...])
        return 0

    lax.fori_loop(0, NBF, blk, 0)


def _tc_encode(part, w1, b1, g1, be1, w2, b2, g2, be2, w3, b3):
    f2, m2, v2 = pl.pallas_call(
        _tc1_body,
        out_shape=[
            jax.ShapeDtypeStruct((64, TOTW), jnp.float32),
            jax.ShapeDtypeStruct((64, 1), jnp.float32),
            jax.ShapeDtypeStruct((64, 1), jnp.float32),
        ],
        scratch_shapes=[
            pltpu.VMEM((5, COLS), jnp.float32),
            pltpu.VMEM((32, COLS), jnp.float32),
        ],
    )(part, w1, g1, be1, w2)
    return pl.pallas_call(
        _tc2_body,
        out_shape=jax.ShapeDtypeStruct((128, TOTW), jnp.float32),
    )(f2, g2, be2, m2, v2, w3, b3)


def kernel(points, w1, b1, g1, be1, w2, b2, g2, be2, w3, b3):
    pts = points.T.reshape(4, NG, PPG)
    pts = jnp.pad(pts, ((0, 0), (0, 0), (0, PPGP - PPG)))
    pts = pts.transpose(1, 0, 2)

    part = _sc_build(pts)

    out = _tc_encode(
        part,
        w1.transpose(2, 3, 0, 1),
        b1.reshape(32, 1), g1.reshape(32, 1), be1.reshape(32, 1),
        w2.transpose(2, 3, 0, 1),
        b2.reshape(64, 1), g2.reshape(64, 1), be2.reshape(64, 1),
        w3.reshape(128, 64), b3.reshape(128, 1))
    return out[:, :HW].reshape(1, 128, BEV_H, BEV_W)
